# baseline (speedup 1.0000x reference)
import dataclasses
import functools

import jax
import jax.numpy as jnp
import numpy as np
from jax import lax
from jax.experimental import pallas as pl
from jax.experimental.pallas import tpu as pltpu
from jax.experimental.pallas import tpu_sc as plsc

N_ROWS = 128
N_COLS = 32768
K = 64
NC = 2
NS = 16
NW = NC * NS
RPW = N_ROWS // NW
NV = N_COLS // 16
SIGN = np.int32(-2147483648)
MAGN = np.int32(0x7FFFFFFF)


def _key(x):
    u = plsc.bitcast(x, jnp.int32)
    s = lax.shift_right_arithmetic(u, 31)
    return lax.bitwise_xor(u, lax.bitwise_and(s, MAGN))


def _topk_body(x_hbm, vals_hbm, idx_hbm,
               raw_v, cand_v, hist_v, outv_v, outi_v,
               sk_v, sp_v, sortv_v, sorti_v, ks_v, ps_v, cnt_v, nid_v):
    cid = lax.axis_index("c")
    sid = lax.axis_index("s")
    wid = sid * NC + cid
    lanes = lax.iota(jnp.int32, 16)
    zeros16 = lanes - lanes
    ones = zeros16 + 1

    def zero_hist():
        @plsc.parallel_loop(0, 256, unroll=8)
        def _z(i):
            hist_v[pl.ds(i * 16, 16)] = zeros16

    def sweep(r_splat):
        @plsc.parallel_loop(0, 256, unroll=8, carry=zeros16)
        def csum(w, c):
            return c + plsc.load_gather(hist_v, [lanes * 256 + w])

        sfx = lax.rev(plsc.cumsum(lax.rev(csum, (0,))), (0,))
        ge = lax.rev((sfx >= r_splat).astype(jnp.int32), (0,))
        gc = 15 - plsc.all_reduce_ffs(ge == 1)
        tot = zeros16
        base_idx = (gc * 16 + lanes) * 16
        for l in range(16):
            tot = tot + plsc.load_gather(hist_v, [base_idx + l])
        sk_v[...] = sfx
        above = jnp.where(gc >= 15, zeros16,
                          plsc.load_gather(sk_v, [jnp.minimum(gc + 1,
                                                              zeros16 + 15)]))
        bsfx = lax.rev(plsc.cumsum(lax.rev(tot, (0,))), (0,)) + above
        geb = lax.rev((bsfx >= r_splat).astype(jnp.int32), (0,))
        jb = 15 - plsc.all_reduce_ffs(geb == 1)
        b = gc * 16 + jb
        sk_v[...] = tot
        t = plsc.load_gather(sk_v, [jb])
        sp_v[...] = bsfx
        g = plsc.load_gather(sp_v, [jb]) - t
        return b, g, t

    @pl.loop(0, RPW)
    def _row(t):
        row = wid * RPW + t
        pltpu.sync_copy(x_hbm.at[row], raw_v)

        zero_hist()

        lane15 = lanes == 15

        @plsc.parallel_loop(0, NV, unroll=8)
        def _s1(i):
            x = raw_v[pl.ds(i * 16, 16)]
            kb = lax.bitwise_xor(_key(x), SIGN)
            d = lax.shift_right_logical(kb, 24)
            plsc.addupdate_scatter(hist_v, [d * 16 + lanes], ones)
            cm = plsc.cummax(lax.bitwise_xor(kb, SIGN))
            plsc.store_scatter(cnt_v, [zeros16 + i], cm, mask=lane15)

        b1, g1, t1 = sweep(zeros16 + K)

        @plsc.parallel_loop(0, NV // 16, unroll=4, carry=zeros16)
        def nout(g, nout_c):
            mk = lax.bitwise_xor(cnt_v[pl.ds(g * 16, 16)], SIGN)
            m2 = lax.shift_right_logical(mk, 24) >= b1
            posn = nout_c + plsc.cumsum(m2.astype(jnp.int32)) - 1
            plsc.store_scatter(nid_v, [posn], g * 16 + lanes, mask=m2)
            return nout_c + plsc.all_reduce_population_count(m2)

        nn = jnp.max(nout)

        @plsc.parallel_loop(0, nn, unroll=2, carry=zeros16)
        def run(q, run_c):
            jsp = plsc.load_gather(nid_v, [zeros16 + q])
            idxv = jsp * 16 + lanes
            x = plsc.load_gather(raw_v, [idxv])
            kb = lax.bitwise_xor(_key(x), SIGN)
            d = lax.shift_right_logical(kb, 24)
            m = d >= b1
            pos = run_c + plsc.cumsum(m.astype(jnp.int32)) - 1
            plsc.store_scatter(cand_v, [pos], idxv, mask=m)
            return run_c + plsc.all_reduce_population_count(m)

        n = jnp.max(run)
        nvc = lax.shift_right_logical(n + 15, 4)

        def refine(B, r, cnt, shift):
            zero_hist()

            @plsc.parallel_loop(0, nvc, unroll=2)
            def _rb(i):
                base = i * 16
                valid = (base + lanes) < n
                cidx = cand_v[pl.ds(base, 16)]
                x = plsc.load_gather(raw_v, [cidx], mask=valid)
                kb = lax.bitwise_xor(_key(x), SIGN)
                pref = lax.shift_right_logical(kb, shift + 8)
                tie = jnp.logical_and(pref == B, valid)
                d = lax.bitwise_and(
                    lax.shift_right_logical(kb, shift), np.int32(0xFF))
                plsc.addupdate_scatter(hist_v, [d * 16 + lanes], ones, mask=tie)
            b, g, tb = sweep(r)
            return B * 256 + b, r - g, tb

        B, r, cnt = b1, (zeros16 + K) - g1, t1
        for shift in (16, 8, 0):
            def _skip(B, r, cnt):
                return B * 256, r, cnt

            def _do(B, r, cnt, _s=shift):
                return refine(B, r, cnt, _s)

            B, r, cnt = lax.cond(jnp.min(jnp.where(r == cnt, ones, zeros16)) == 1,
                              _skip, _do, B, r, cnt)

        take_all = r == cnt
        ts = lax.bitwise_xor(B, SIGN)
        tcut = jnp.where(take_all, ts - 1, ts)
        r_tie = jnp.where(take_all, 0, r)

        @plsc.parallel_loop(0, nvc, unroll=2, carry=(zeros16, zeros16))
        def _fin(i, carry):
            off_o, eq_seen = carry
            base = i * 16
            valid = (base + lanes) < n
            cidx = cand_v[pl.ds(base, 16)]
            x = plsc.load_gather(raw_v, [cidx], mask=valid)
            ms = _key(x)
            gt = jnp.logical_and(ms > tcut, valid)
            eq = jnp.logical_and(jnp.logical_and(ms == ts, valid),
                                 jnp.logical_not(gt))
            eq_rank = eq_seen + plsc.cumsum(eq.astype(jnp.int32))
            sel = jnp.logical_or(gt, jnp.logical_and(eq, eq_rank <= r_tie))
            pos = off_o + plsc.cumsum(sel.astype(jnp.int32)) - 1
            plsc.store_scatter(outv_v, [pos], x, mask=sel)
            plsc.store_scatter(outi_v, [pos], cidx, mask=sel)
            return (off_o + plsc.all_reduce_population_count(sel),
                    eq_seen + plsc.all_reduce_population_count(eq))

        def ce(ka, pa, kb, pb):
            c = ka >= kb
            return (jnp.where(c, ka, kb), jnp.where(c, pa, pb),
                    jnp.where(c, kb, ka), jnp.where(c, pb, pa))

        def rev(k, p):
            return lax.rev(k, (0,)), lax.rev(p, (0,))

        def perm_gather(k, p, pidx):
            sk_v[...] = k
            sp_v[...] = p
            return (plsc.load_gather(sk_v, [pidx]),
                    plsc.load_gather(sp_v, [pidx]))

        def clean(k, p):
            for j in (8, 4, 2, 1):
                pidx = lax.bitwise_xor(lanes, np.int32(j))
                kp, pp = perm_gather(k, p, pidx)
                is_lo = lax.bitwise_and(lanes, np.int32(j)) == 0
                keep = jnp.where(is_lo, k >= kp, kp >= k)
                k = jnp.where(keep, k, kp)
                p = jnp.where(keep, p, pp)
            return k, p

        def merge32(ka, pa, kb, pb):
            kb, pb = rev(kb, pb)
            ka, pa, kb, pb = ce(ka, pa, kb, pb)
            ka, pa = clean(ka, pa)
            kb, pb = clean(kb, pb)
            return ka, pa, kb, pb

        ks, ps = [], []
        for v in range(4):
            xv = outv_v[pl.ds(v * 16, 16)]
            kv, pv = plsc.sort_key_val(_key(xv), v * 16 + lanes,
                                       descending=True)
            ks.append(kv)
            ps.append(pv)

        ks[0], ps[0], ks[1], ps[1] = merge32(ks[0], ps[0], ks[1], ps[1])
        ks[2], ps[2], ks[3], ps[3] = merge32(ks[2], ps[2], ks[3], ps[3])

        rk3, rp3 = rev(ks[3], ps[3])
        rk2, rp2 = rev(ks[2], ps[2])
        k0, p0, rk3, rp3 = ce(ks[0], ps[0], rk3, rp3)
        k1, p1, rk2, rp2 = ce(ks[1], ps[1], rk2, rp2)
        k0, p0, k1, p1 = ce(k0, p0, k1, p1)
        rk3, rp3, rk2, rp2 = ce(rk3, rp3, rk2, rp2)
        k0, p0 = clean(k0, p0)
        k1, p1 = clean(k1, p1)
        k2, p2 = clean(rk3, rp3)
        k3, p3 = clean(rk2, rp2)

        kall = (k0, k1, k2, k3)
        for v in range(4):
            ks_v[pl.ds(v * 16, 16)] = kall[v]
        ps = [p0, p1, p2, p3]
        for q in (0, 1, 0, 1):
            for v in range(4):
                ps_v[pl.ds(v * 16, 16)] = ps[v]
            new_ps = []
            for v in range(4):
                e = v * 16 + lanes
                if q == 0:
                    partner = lax.bitwise_xor(e, 1)
                else:
                    partner = jnp.clip(lax.bitwise_xor(e + 1, 1) - 1, 0, 63)
                kp = plsc.load_gather(ks_v, [partner])
                pp = plsc.load_gather(ps_v, [partner])
                k, p = kall[v], ps[v]
                take = jnp.logical_and(
                    k == kp,
                    jnp.where(partner > e, pp < p, pp > p))
                new_ps.append(jnp.where(take, pp, p))
            ps = new_ps

        for v, pv in enumerate(ps):
            sortv_v[pl.ds(v * 16, 16)] = plsc.load_gather(outv_v, [pv])
            sorti_v[pl.ds(v * 16, 16)] = plsc.load_gather(outi_v, [pv])

        pltpu.sync_copy(sortv_v, vals_hbm.at[row])
        pltpu.sync_copy(sorti_v, idx_hbm.at[row])


@jax.jit
def _sc_topk(inputs):
    mesh = plsc.VectorSubcoreMesh(core_axis_name="c", subcore_axis_name="s")
    cp = pltpu.CompilerParams()
    if "needs_layout_passes" in pltpu.CompilerParams.__dataclass_fields__:
        cp = dataclasses.replace(cp, needs_layout_passes=False)
    f = pl.kernel(
        _topk_body,
        compiler_params=cp,
        out_type=[
            jax.ShapeDtypeStruct((N_ROWS, K), jnp.float32),
            jax.ShapeDtypeStruct((N_ROWS, K), jnp.int32),
        ],
        mesh=mesh,
        scratch_types=[
            pltpu.VMEM((N_COLS,), jnp.float32),
            pltpu.VMEM((N_COLS,), jnp.int32),
            pltpu.VMEM((256 * 16,), jnp.int32),
            pltpu.VMEM((K,), jnp.float32),
            pltpu.VMEM((K,), jnp.int32),
            pltpu.VMEM((16,), jnp.int32),
            pltpu.VMEM((16,), jnp.int32),
            pltpu.VMEM((K,), jnp.float32),
            pltpu.VMEM((K,), jnp.int32),
            pltpu.VMEM((K,), jnp.int32),
            pltpu.VMEM((K,), jnp.int32),
            pltpu.VMEM((NV,), jnp.int32),
            pltpu.VMEM((NV,), jnp.int32),
        ],
    )
    return f(inputs)


def kernel(inputs):
    vals, idxs = _sc_topk(inputs)
    return (vals, idxs)


# speedup vs baseline: 1.2050x; 1.2050x over previous
import dataclasses
import functools

import jax
import jax.numpy as jnp
import numpy as np
from jax import lax
from jax.experimental import pallas as pl
from jax.experimental.pallas import tpu as pltpu
from jax.experimental.pallas import tpu_sc as plsc

N_ROWS = 128
N_COLS = 32768
K = 64
NC = 2
NS = 16
NW = NC * NS
RPW = N_ROWS // NW
NV = N_COLS // 16
SIGN = np.int32(-2147483648)
MAGN = np.int32(0x7FFFFFFF)


def _key(x):
    u = plsc.bitcast(x, jnp.int32)
    s = lax.shift_right_arithmetic(u, 31)
    return lax.bitwise_xor(u, lax.bitwise_and(s, MAGN))


def _topk_body(x_hbm, vals_hbm, idx_hbm,
               raw_v, cand_v, hist_v, outv_v, outi_v,
               sk_v, sp_v, sortv_v, sorti_v, ks_v, ps_v, cnt_v, nid_v):
    cid = lax.axis_index("c")
    sid = lax.axis_index("s")
    wid = sid * NC + cid
    lanes = lax.iota(jnp.int32, 16)
    zeros16 = lanes - lanes
    ones = zeros16 + 1

    def zero_hist():
        @plsc.parallel_loop(0, 256, unroll=8)
        def _z(i):
            hist_v[pl.ds(i * 16, 16)] = zeros16

    def sweep(r_splat):
        @plsc.parallel_loop(0, 256, unroll=8, carry=zeros16)
        def csum(w, c):
            ws = lax.bitwise_and(zeros16 + w + lanes, zeros16 + 255)
            return c + plsc.load_gather(hist_v, [lanes * 256 + ws])

        sfx = lax.rev(plsc.cumsum(lax.rev(csum, (0,))), (0,))
        ge = lax.rev((sfx >= r_splat).astype(jnp.int32), (0,))
        gc = 15 - plsc.all_reduce_ffs(ge == 1)
        tot = zeros16
        base_idx = (gc * 16 + lanes) * 16
        for l in range(16):
            ls = lax.bitwise_and(lanes + l, zeros16 + 15)
            tot = tot + plsc.load_gather(hist_v, [base_idx + ls])
        sk_v[...] = sfx
        above = jnp.where(gc >= 15, zeros16,
                          plsc.load_gather(sk_v, [jnp.minimum(gc + 1,
                                                              zeros16 + 15)]))
        bsfx = lax.rev(plsc.cumsum(lax.rev(tot, (0,))), (0,)) + above
        geb = lax.rev((bsfx >= r_splat).astype(jnp.int32), (0,))
        jb = 15 - plsc.all_reduce_ffs(geb == 1)
        b = gc * 16 + jb
        sk_v[...] = tot
        t = plsc.load_gather(sk_v, [jb])
        sp_v[...] = bsfx
        g = plsc.load_gather(sp_v, [jb]) - t
        return b, g, t

    @pl.loop(0, RPW)
    def _row(t):
        row = wid * RPW + t
        pltpu.sync_copy(x_hbm.at[row], raw_v)

        zero_hist()

        lane15 = lanes == 15

        @plsc.parallel_loop(0, NV, unroll=8)
        def _s1(i):
            x = raw_v[pl.ds(i * 16, 16)]
            kb = lax.bitwise_xor(_key(x), SIGN)
            d = lax.shift_right_logical(kb, 24)
            plsc.addupdate_scatter(hist_v, [d * 16 + lanes], ones)
            cm = plsc.cummax(lax.bitwise_xor(kb, SIGN))
            plsc.store_scatter(cnt_v, [zeros16 + i], cm, mask=lane15)

        b1, g1, t1 = sweep(zeros16 + K)

        @plsc.parallel_loop(0, NV // 16, unroll=4, carry=zeros16)
        def nout(g, nout_c):
            mk = lax.bitwise_xor(cnt_v[pl.ds(g * 16, 16)], SIGN)
            m2 = lax.shift_right_logical(mk, 24) >= b1
            posn = nout_c + plsc.cumsum(m2.astype(jnp.int32)) - 1
            plsc.store_scatter(nid_v, [posn], g * 16 + lanes, mask=m2)
            return nout_c + plsc.all_reduce_population_count(m2)

        nn = jnp.max(nout)

        @plsc.parallel_loop(0, nn, unroll=2, carry=zeros16)
        def run(q, run_c):
            jsp = plsc.load_gather(nid_v, [zeros16 + q])
            idxv = jsp * 16 + lanes
            x = plsc.load_gather(raw_v, [idxv])
            kb = lax.bitwise_xor(_key(x), SIGN)
            d = lax.shift_right_logical(kb, 24)
            m = d >= b1
            pos = run_c + plsc.cumsum(m.astype(jnp.int32)) - 1
            plsc.store_scatter(cand_v, [pos], idxv, mask=m)
            return run_c + plsc.all_reduce_population_count(m)

        n = jnp.max(run)
        nvc = lax.shift_right_logical(n + 15, 4)

        def refine(B, r, cnt, shift):
            zero_hist()

            @plsc.parallel_loop(0, nvc, unroll=2)
            def _rb(i):
                base = i * 16
                valid = (base + lanes) < n
                cidx = cand_v[pl.ds(base, 16)]
                x = plsc.load_gather(raw_v, [cidx], mask=valid)
                kb = lax.bitwise_xor(_key(x), SIGN)
                pref = lax.shift_right_logical(kb, shift + 8)
                tie = jnp.logical_and(pref == B, valid)
                d = lax.bitwise_and(
                    lax.shift_right_logical(kb, shift), np.int32(0xFF))
                plsc.addupdate_scatter(hist_v, [d * 16 + lanes], ones, mask=tie)
            b, g, tb = sweep(r)
            return B * 256 + b, r - g, tb

        B, r, cnt = b1, (zeros16 + K) - g1, t1
        for shift in (16, 8, 0):
            def _skip(B, r, cnt):
                return B * 256, r, cnt

            def _do(B, r, cnt, _s=shift):
                return refine(B, r, cnt, _s)

            B, r, cnt = lax.cond(jnp.min(jnp.where(r == cnt, ones, zeros16)) == 1,
                              _skip, _do, B, r, cnt)

        take_all = r == cnt
        ts = lax.bitwise_xor(B, SIGN)
        tcut = jnp.where(take_all, ts - 1, ts)
        r_tie = jnp.where(take_all, 0, r)

        @plsc.parallel_loop(0, nvc, unroll=2, carry=(zeros16, zeros16))
        def _fin(i, carry):
            off_o, eq_seen = carry
            base = i * 16
            valid = (base + lanes) < n
            cidx = cand_v[pl.ds(base, 16)]
            x = plsc.load_gather(raw_v, [cidx], mask=valid)
            ms = _key(x)
            gt = jnp.logical_and(ms > tcut, valid)
            eq = jnp.logical_and(jnp.logical_and(ms == ts, valid),
                                 jnp.logical_not(gt))
            eq_rank = eq_seen + plsc.cumsum(eq.astype(jnp.int32))
            sel = jnp.logical_or(gt, jnp.logical_and(eq, eq_rank <= r_tie))
            pos = off_o + plsc.cumsum(sel.astype(jnp.int32)) - 1
            plsc.store_scatter(outv_v, [pos], x, mask=sel)
            plsc.store_scatter(outi_v, [pos], cidx, mask=sel)
            return (off_o + plsc.all_reduce_population_count(sel),
                    eq_seen + plsc.all_reduce_population_count(eq))

        def ce(ka, pa, kb, pb):
            c = ka >= kb
            return (jnp.where(c, ka, kb), jnp.where(c, pa, pb),
                    jnp.where(c, kb, ka), jnp.where(c, pb, pa))

        def rev(k, p):
            return lax.rev(k, (0,)), lax.rev(p, (0,))

        def perm_gather(k, p, pidx):
            sk_v[...] = k
            sp_v[...] = p
            return (plsc.load_gather(sk_v, [pidx]),
                    plsc.load_gather(sp_v, [pidx]))

        def clean(k, p):
            for j in (8, 4, 2, 1):
                pidx = lax.bitwise_xor(lanes, np.int32(j))
                kp, pp = perm_gather(k, p, pidx)
                is_lo = lax.bitwise_and(lanes, np.int32(j)) == 0
                keep = jnp.where(is_lo, k >= kp, kp >= k)
                k = jnp.where(keep, k, kp)
                p = jnp.where(keep, p, pp)
            return k, p

        def merge32(ka, pa, kb, pb):
            kb, pb = rev(kb, pb)
            ka, pa, kb, pb = ce(ka, pa, kb, pb)
            ka, pa = clean(ka, pa)
            kb, pb = clean(kb, pb)
            return ka, pa, kb, pb

        ks, ps = [], []
        for v in range(4):
            xv = outv_v[pl.ds(v * 16, 16)]
            kv, pv = plsc.sort_key_val(_key(xv), v * 16 + lanes,
                                       descending=True)
            ks.append(kv)
            ps.append(pv)

        ks[0], ps[0], ks[1], ps[1] = merge32(ks[0], ps[0], ks[1], ps[1])
        ks[2], ps[2], ks[3], ps[3] = merge32(ks[2], ps[2], ks[3], ps[3])

        rk3, rp3 = rev(ks[3], ps[3])
        rk2, rp2 = rev(ks[2], ps[2])
        k0, p0, rk3, rp3 = ce(ks[0], ps[0], rk3, rp3)
        k1, p1, rk2, rp2 = ce(ks[1], ps[1], rk2, rp2)
        k0, p0, k1, p1 = ce(k0, p0, k1, p1)
        rk3, rp3, rk2, rp2 = ce(rk3, rp3, rk2, rp2)
        k0, p0 = clean(k0, p0)
        k1, p1 = clean(k1, p1)
        k2, p2 = clean(rk3, rp3)
        k3, p3 = clean(rk2, rp2)

        kall = (k0, k1, k2, k3)
        for v in range(4):
            ks_v[pl.ds(v * 16, 16)] = kall[v]
        ps = [p0, p1, p2, p3]
        for q in (0, 1, 0, 1):
            for v in range(4):
                ps_v[pl.ds(v * 16, 16)] = ps[v]
            new_ps = []
            for v in range(4):
                e = v * 16 + lanes
                if q == 0:
                    partner = lax.bitwise_xor(e, 1)
                else:
                    partner = jnp.clip(lax.bitwise_xor(e + 1, 1) - 1, 0, 63)
                kp = plsc.load_gather(ks_v, [partner])
                pp = plsc.load_gather(ps_v, [partner])
                k, p = kall[v], ps[v]
                take = jnp.logical_and(
                    k == kp,
                    jnp.where(partner > e, pp < p, pp > p))
                new_ps.append(jnp.where(take, pp, p))
            ps = new_ps

        for v, pv in enumerate(ps):
            sortv_v[pl.ds(v * 16, 16)] = plsc.load_gather(outv_v, [pv])
            sorti_v[pl.ds(v * 16, 16)] = plsc.load_gather(outi_v, [pv])

        pltpu.sync_copy(sortv_v, vals_hbm.at[row])
        pltpu.sync_copy(sorti_v, idx_hbm.at[row])


@jax.jit
def _sc_topk(inputs):
    mesh = plsc.VectorSubcoreMesh(core_axis_name="c", subcore_axis_name="s")
    cp = pltpu.CompilerParams()
    if "needs_layout_passes" in pltpu.CompilerParams.__dataclass_fields__:
        cp = dataclasses.replace(cp, needs_layout_passes=False)
    f = pl.kernel(
        _topk_body,
        compiler_params=cp,
        out_type=[
            jax.ShapeDtypeStruct((N_ROWS, K), jnp.float32),
            jax.ShapeDtypeStruct((N_ROWS, K), jnp.int32),
        ],
        mesh=mesh,
        scratch_types=[
            pltpu.VMEM((N_COLS,), jnp.float32),
            pltpu.VMEM((N_COLS,), jnp.int32),
            pltpu.VMEM((256 * 16,), jnp.int32),
            pltpu.VMEM((K,), jnp.float32),
            pltpu.VMEM((K,), jnp.int32),
            pltpu.VMEM((16,), jnp.int32),
            pltpu.VMEM((16,), jnp.int32),
            pltpu.VMEM((K,), jnp.float32),
            pltpu.VMEM((K,), jnp.int32),
            pltpu.VMEM((K,), jnp.int32),
            pltpu.VMEM((K,), jnp.int32),
            pltpu.VMEM((NV,), jnp.int32),
            pltpu.VMEM((NV,), jnp.int32),
        ],
    )
    return f(inputs)


def kernel(inputs):
    vals, idxs = _sc_topk(inputs)
    return (vals, idxs)
